# baseline (speedup 1.0000x reference)
import functools

import jax
import jax.numpy as jnp
from jax import lax
from jax.experimental import pallas as pl
from jax.experimental.pallas import tpu as pltpu
from jax.experimental.pallas import tpu_sc as plsc

VOCAB = 1000000
D = 64
DP = 2 * D
FC_OUT = 128
B = 4096
L = 200
NC = 2
NS = 16
NW = NC * NS
SPW = B // NW
TPW = SPW * L
CHUNK0 = 128
CHUNK1 = L - CHUNK0


def _sc_body(idx1_hbm, idx2_hbm, table_hbm, out1_hbm, out2_hbm,
             idx_v, pair_v, rows_v, out_v, sem0, sem1):
    wid = lax.axis_index("s") * NC + lax.axis_index("c")
    base_tok = wid * TPW
    sems = (sem0, sem1)

    def gather(s, b, sem, start):
        off = s * L
        mk = pltpu.async_copy if start else (
            lambda src, dst, sm: pltpu.make_async_copy(src, dst, sm).wait())
        mk(table_hbm.at[pair_v.at[pl.ds(off, CHUNK0)]],
           rows_v.at[b, pl.ds(0, CHUNK0), :], sem)
        mk(table_hbm.at[pair_v.at[pl.ds(off + CHUNK0, CHUNK1)]],
           rows_v.at[b, pl.ds(CHUNK0, CHUNK1), :], sem)

    for idx_hbm, out_hbm in ((idx1_hbm, out1_hbm), (idx2_hbm, out2_hbm)):
        pltpu.sync_copy(idx_hbm.at[pl.ds(base_tok, TPW)], idx_v)

        def shift_body(i, carry):
            pair_v[pl.ds(i * 16, 16)] = jnp.right_shift(
                idx_v[pl.ds(i * 16, 16)], 1)
            return carry

        lax.fori_loop(0, TPW // 16, shift_body, 0, unroll=8)

        gather(0, 0, sem0, True)
        gather(1, 1, sem1, True)

        def blk_body(i, carry):
            for b in range(2):
                s = 2 * i + b
                gather(s, b, sems[b], False)
                off = s * L

                def add_tok(accs, iv, j, t):
                    half = (iv[j] & 1) * D
                    return tuple(
                        accs[k] + rows_v[b, t, pl.ds(half + k * 16, 16)]
                        for k in range(4))

                def grp_body(g, accs):
                    iv = idx_v[pl.ds(off + g * 16, 16)]
                    for j in range(16):
                        accs = add_tok(accs, iv, j, g * 16 + j)
                    return accs

                accs = lax.fori_loop(
                    0, L // 16, grp_body,
                    tuple(jnp.zeros((16,), jnp.float32) for _ in range(4)))
                iv = idx_v[pl.ds(off + L - 16, 16)]
                for j in range(8, 16):
                    accs = add_tok(accs, iv, j, L - 16 + j)
                for k in range(4):
                    out_v[s, pl.ds(k * 16, 16)] = accs[k] * (1.0 / L)

                ns = s + 2

                @pl.when(ns < SPW)
                def _():
                    gather(ns, b, sems[b], True)
            return carry

        lax.fori_loop(0, SPW // 2, blk_body, 0)
        pltpu.sync_copy(out_v, out_hbm.at[pl.ds(wid * SPW, SPW), :])


_sc_means = pl.kernel(
    _sc_body,
    out_type=(jax.ShapeDtypeStruct((B, D), jnp.float32),
              jax.ShapeDtypeStruct((B, D), jnp.float32)),
    mesh=plsc.VectorSubcoreMesh(core_axis_name="c", subcore_axis_name="s"),
    compiler_params=pltpu.CompilerParams(use_tc_tiling_on_sc=True),
    scratch_types=[
        pltpu.VMEM((TPW,), jnp.int32),
        pltpu.VMEM((TPW,), jnp.int32),
        pltpu.VMEM((2, L, DP), jnp.float32),
        pltpu.VMEM((SPW, D), jnp.float32),
        pltpu.SemaphoreType.DMA,
        pltpu.SemaphoreType.DMA,
    ],
)


def _mm_body(x_ref, w_ref, o_ref):
    o_ref[:, :] = lax.dot_general(
        x_ref[:, :], w_ref[:, :],
        (((1,), (1,)), ((), ())),
        preferred_element_type=jnp.float32)


def _head(x, w):
    return pl.pallas_call(
        _mm_body,
        out_shape=jax.ShapeDtypeStruct((B, FC_OUT), jnp.float32),
    )(x, w)


def kernel(inputs_1, inputs_2, ebd_table, fc_weight):
    idx1 = inputs_1.reshape(-1).astype(jnp.int32)
    idx2 = inputs_2.reshape(-1).astype(jnp.int32)
    table2 = ebd_table.reshape(VOCAB // 2, DP)
    mean1, mean2 = _sc_means(idx1, idx2, table2)
    out1 = _head(mean1, fc_weight)
    out2 = _head(mean2, fc_weight)
    return (out1, out2)


# speedup vs baseline: 1.2127x; 1.2127x over previous
import functools

import jax
import jax.numpy as jnp
from jax import lax
from jax.experimental import pallas as pl
from jax.experimental.pallas import tpu as pltpu
from jax.experimental.pallas import tpu_sc as plsc

VOCAB = 1000000
D = 64
FC_OUT = 128
B = 4096
L = 200
NC = 2
NS = 16
NW = NC * NS
SPW = B // NW
TPW = SPW * L
NBUF = 2
NGRP = L // 16


def _sc_body(idx1_hbm, idx2_hbm, table_hbm, out1_hbm, out2_hbm,
             idx_v, rows_v, out_v, *sems):
    wid = lax.axis_index("s") * NC + lax.axis_index("c")
    base_tok = wid * TPW

    def enq_rows(iv, j0, nb, t0):
        for j in range(j0, 16):
            pltpu.async_copy(table_hbm.at[pl.ds(iv[j], 1), :],
                             rows_v.at[nb, pl.ds(t0 + j, 1), :], sems[nb])

    def enqueue_sentence(ns, nb):
        def g_body(g, carry):
            iv = idx_v[pl.ds(ns * L + g * 16, 16)]
            enq_rows(iv, 0, nb, g * 16)
            return carry

        lax.fori_loop(0, NGRP, g_body, 0)
        iv = idx_v[pl.ds(ns * L + (L - 16), 16)]
        enq_rows(iv, 8, nb, L - 16)

    def drain(nb):
        pltpu.make_async_copy(table_hbm.at[pl.ds(0, L), :],
                              rows_v.at[nb], sems[nb]).wait()

    for idx_hbm, out_hbm in ((idx1_hbm, out1_hbm), (idx2_hbm, out2_hbm)):
        pltpu.sync_copy(idx_hbm.at[pl.ds(base_tok, TPW)], idx_v)

        for ps in range(NBUF - 1):
            enqueue_sentence(ps, ps)

        def blk_body(i, carry):
            for bb in range(NBUF):
                s = NBUF * i + bb
                ns = lax.rem(s + NBUF - 1, SPW)
                nb = (bb + NBUF - 1) % NBUF
                drain(bb)

                def grp_body(g, accs):
                    iv = idx_v[pl.ds(ns * L + g * 16, 16)]
                    enq_rows(iv, 0, nb, g * 16)
                    t0 = g * 16
                    for j in range(16):
                        accs = tuple(
                            accs[k] + rows_v[bb, t0 + j, pl.ds(k * 16, 16)]
                            for k in range(4))
                    return accs

                accs = lax.fori_loop(
                    0, NGRP, grp_body,
                    tuple(jnp.zeros((16,), jnp.float32) for _ in range(4)))
                iv = idx_v[pl.ds(ns * L + (L - 16), 16)]
                enq_rows(iv, 8, nb, L - 16)
                for j in range(8, 16):
                    accs = tuple(
                        accs[k] + rows_v[bb, L - 16 + j, pl.ds(k * 16, 16)]
                        for k in range(4))
                for k in range(4):
                    out_v[s, pl.ds(k * 16, 16)] = accs[k] * (1.0 / L)
            return carry

        lax.fori_loop(0, SPW // NBUF, blk_body, 0)
        for nb in range(NBUF - 1):
            drain(nb)
        pltpu.sync_copy(out_v, out_hbm.at[pl.ds(wid * SPW, SPW), :])


_sc_means = pl.kernel(
    _sc_body,
    out_type=(jax.ShapeDtypeStruct((B, D), jnp.float32),
              jax.ShapeDtypeStruct((B, D), jnp.float32)),
    mesh=plsc.VectorSubcoreMesh(core_axis_name="c", subcore_axis_name="s"),
    compiler_params=pltpu.CompilerParams(use_tc_tiling_on_sc=True),
    scratch_types=[
        pltpu.VMEM((TPW,), jnp.int32),
        pltpu.VMEM((NBUF, L, D), jnp.float32),
        pltpu.VMEM((SPW, D), jnp.float32),
    ] + [pltpu.SemaphoreType.DMA] * NBUF,
)


def _mm_body(x_ref, w_ref, o_ref):
    o_ref[:, :] = lax.dot_general(
        x_ref[:, :], w_ref[:, :],
        (((1,), (1,)), ((), ())),
        preferred_element_type=jnp.float32)


def _head(x, w):
    return pl.pallas_call(
        _mm_body,
        out_shape=jax.ShapeDtypeStruct((B, FC_OUT), jnp.float32),
    )(x, w)


def kernel(inputs_1, inputs_2, ebd_table, fc_weight):
    idx1 = inputs_1.reshape(-1).astype(jnp.int32)
    idx2 = inputs_2.reshape(-1).astype(jnp.int32)
    mean1, mean2 = _sc_means(idx1, idx2, ebd_table)
    out1 = _head(mean1, fc_weight)
    out2 = _head(mean2, fc_weight)
    return (out1, out2)
